# baseline (speedup 1.0000x reference)
import functools

import jax
import jax.numpy as jnp
from jax import lax
from jax.experimental import pallas as pl
from jax.experimental.pallas import tpu as pltpu
from jax.experimental.pallas import tpu_sc as plsc

B, V, D, P = 1024, 100000, 16, 20
_NROWS = V - 1

_NC, _NS = 2, 16
_NW = _NC * _NS
_BP = B * P
_NE = D * _BP
_EW = _NE // _NW


def _sc_gather(table1d, eidx):
    mesh = plsc.VectorSubcoreMesh(core_axis_name="c", subcore_axis_name="s")

    @functools.partial(
        pl.kernel,
        out_type=jax.ShapeDtypeStruct((_NE,), jnp.float32),
        mesh=mesh,
        scratch_types=[
            pltpu.VMEM((_EW,), jnp.int32),
            pltpu.VMEM((_EW,), jnp.float32),
            pltpu.SemaphoreType.DMA,
        ],
        compiler_params=pltpu.CompilerParams(use_tc_tiling_on_sc=False),
    )
    def k(tab_hbm, eidx_hbm, out_hbm, idx_v, vals_v, sem):
        wid = lax.axis_index("s") * _NC + lax.axis_index("c")
        base = wid * _EW
        pltpu.sync_copy(eidx_hbm.at[pl.ds(base, _EW)], idx_v)
        pltpu.async_copy(tab_hbm.at[idx_v], vals_v, sem).wait()
        pltpu.sync_copy(vals_v, out_hbm.at[pl.ds(base, _EW)])

    return k(table1d, eidx)


_VT = 4096
_NBLK = (V + _VT - 1) // _VT
_VLAST = V - (_NBLK - 1) * _VT


def _mm_body(ivt_ref, w_ref, out_ref):
    i = pl.program_id(0)

    @pl.when(i == 0)
    def _init():
        out_ref[...] = jnp.zeros_like(out_ref)

    def contrib(wb, ab):
        return lax.dot_general(wb, ab, (((1,), (0,)), ((), ())),
                               preferred_element_type=jnp.float32)

    @pl.when(i < _NBLK - 1)
    def _full():
        out_ref[...] += contrib(w_ref[...].astype(jnp.bfloat16),
                                ivt_ref[...].astype(jnp.bfloat16))

    @pl.when(i == _NBLK - 1)
    def _last():
        mv = lax.broadcasted_iota(jnp.int32, (_VT, 1), 0) < _VLAST
        ab = jnp.where(mv, ivt_ref[...], 0.0)
        mw = lax.broadcasted_iota(jnp.int32, (1, _VT), 1) < _VLAST
        wb = jnp.where(mw, w_ref[...], 0.0)
        out_ref[...] += contrib(wb.astype(jnp.bfloat16),
                                ab.astype(jnp.bfloat16))


def _tc_matmul(ivt, W):
    return pl.pallas_call(
        _mm_body,
        grid=(_NBLK,),
        in_specs=[
            pl.BlockSpec((_VT, B), lambda i: (i, 0)),
            pl.BlockSpec((D, _VT), lambda i: (0, i)),
        ],
        out_specs=pl.BlockSpec((D, B), lambda i: (0, 0)),
        out_shape=jax.ShapeDtypeStruct((D, B), jnp.float32),
        compiler_params=pltpu.CompilerParams(
            dimension_semantics=("arbitrary",),
            vmem_limit_bytes=100 * 1024 * 1024,
        ),
    )(ivt, W)


def _ep_body(xt_ref, pvt_ref, hct_ref, out_ref):
    xt = xt_ref[...]
    logits = jnp.zeros((P, B), jnp.float32)
    for d in range(D):
        logits = logits + pvt_ref[d] * xt[d:d + 1, :]
    t = hct_ref[...].astype(jnp.float32)
    bce = (jnp.maximum(logits, 0.0) - logits * t
           + jnp.log1p(jnp.exp(-jnp.abs(logits))))
    out_ref[0, 0] = jnp.sum(bce) * (1.0 / (B * P))


def _tc_epilogue(xt, pvt, hct):
    return pl.pallas_call(
        _ep_body,
        out_specs=pl.BlockSpec(memory_space=pltpu.SMEM),
        out_shape=jax.ShapeDtypeStruct((1, 1), jnp.float32),
    )(xt, pvt, hct)


def kernel(inputs_vector, path_nodes_indices, huffman_codes, W, cls):
    idx = path_nodes_indices.astype(jnp.int32).T.reshape(_BP)
    eidx = (jnp.arange(D, dtype=jnp.int32)[:, None] * _NROWS
            + idx[None, :]).reshape(_NE)
    table1d = cls.T.reshape(_NROWS * D)
    vals = _sc_gather(table1d, eidx)
    pvt = vals.reshape(D, P, B)
    ivt = inputs_vector.T
    hct = huffman_codes.astype(jnp.int32).T
    xt = _tc_matmul(ivt, W)
    loss = _tc_epilogue(xt, pvt, hct)
    return loss.reshape(1)
